# Initial kernel scaffold; baseline (speedup 1.0000x reference)
import jax
import jax.numpy as jnp
from jax.experimental import pallas as pl


def kernel(x, edge_index, edge_weight_norm, W1_rel, b1, W1_root, W2_rel, b2, W2_root, W3_rel, b3, W3_root, dec_W1, dec_b1, dec_W2, dec_b2):
    raise NotImplementedError("write your pallas kernel here")


# speedup vs baseline: 4.5326x; 4.5326x over previous
import functools

import jax
import jax.numpy as jnp
from jax import lax
from jax.experimental import pallas as pl
from jax.experimental.pallas import tpu as pltpu
from jax.experimental.pallas import tpu_sc as plsc

N = 10000
E = 320000
D = 128
D_Z = 64

NC = 2
NS = 16
NW = NC * NS

CHUNK = 128
W_EDGES = E // NW
FULL = W_EDGES // CHUNK
PAIRS = FULL // 2
TAIL = W_EDGES - FULL * CHUNK

RCHUNK = 128
N_FULL = N // RCHUNK
TAIL_BASE = N_FULL * RCHUNK
TAIL_ROWS = N - TAIL_BASE
RITERS = -(-(N_FULL + 1) // NS)


def _zero_vmem_rows(buf, nrows, d):
    def body(i, _):
        for k in range(d // 16):
            buf[i, pl.ds(k * 16, 16)] = jnp.zeros((16,), jnp.float32)
        return 0
    lax.fori_loop(0, nrows, body, 0)


def _lane_splat(vec, lane):
    return lax.gather(
        vec,
        jnp.full((16, 1), lane, jnp.int32),
        dimension_numbers=lax.GatherDimensionNumbers(
            offset_dims=(), collapsed_slice_dims=(0,), start_index_map=(0,)),
        slice_sizes=(1,),
        mode=lax.GatherScatterMode.PROMISE_IN_BOUNDS)


def _scale_rows(rows, ew_ref, nedges, d_valid):
    def body(j, _):
        evec = ew_ref[pl.ds(j * 16, 16)]
        for e16 in range(16):
            bvec = _lane_splat(evec, e16)
            row = j * 16 + e16
            for k in range(d_valid // 16):
                sl = pl.ds(k * 16, 16)
                rows[row, sl] = rows[row, sl] * bvec
        return 0
    lax.fori_loop(0, nedges // 16, body, 0)


@functools.cache
def _make_sc_scatter(d_valid):
    mesh = plsc.VectorSubcoreMesh(core_axis_name="c", subcore_axis_name="s",
                                  num_cores=NC, num_subcores=NS)

    @functools.partial(
        pl.kernel,
        out_type=jax.ShapeDtypeStruct((NC, N, D), jnp.float32),
        mesh=mesh,
        scratch_types=[
            pltpu.VMEM((CHUNK,), jnp.int32),
            pltpu.VMEM((CHUNK,), jnp.int32),
            pltpu.VMEM((CHUNK,), jnp.int32),
            pltpu.VMEM((CHUNK,), jnp.int32),
            pltpu.VMEM((CHUNK,), jnp.float32),
            pltpu.VMEM((CHUNK,), jnp.float32),
            pltpu.VMEM((CHUNK, D), jnp.float32),
            pltpu.VMEM((CHUNK, D), jnp.float32),
            pltpu.VMEM((TAIL,), jnp.int32),
            pltpu.VMEM((TAIL,), jnp.int32),
            pltpu.VMEM((TAIL,), jnp.float32),
            pltpu.VMEM_SHARED((N, D), jnp.float32),
            pltpu.SemaphoreType.DMA,
            pltpu.SemaphoreType.DMA,
        ],
    )
    def sc_scatter(g_hbm, src_hbm, dst_hbm, ew_hbm, out_hbm,
                   s0, s1, d0, d1, w0, w1, r0, r1, ts, td, tw,
                   acc_sh, sem0, sem1):
        c = lax.axis_index("c")
        s = lax.axis_index("s")
        wid = s * NC + c
        ebase = wid * W_EDGES

        _zero_vmem_rows(r0, CHUNK, D)
        for q in range(RITERS):
            rid = s + NS * q

            @pl.when(rid < N_FULL)
            def _():
                pltpu.sync_copy(r0, acc_sh.at[pl.ds(rid * RCHUNK, RCHUNK)])

        @pl.when(s == NS - 1)
        def _():
            pltpu.sync_copy(r0.at[pl.ds(0, TAIL_ROWS)],
                            acc_sh.at[pl.ds(TAIL_BASE, TAIL_ROWS)])
        plsc.subcore_barrier()

        def load_idx(cid, sv, dv, wv):
            base = ebase + cid * CHUNK
            pltpu.sync_copy(src_hbm.at[pl.ds(base, CHUNK)], sv)
            pltpu.sync_copy(dst_hbm.at[pl.ds(base, CHUNK)], dv)
            pltpu.sync_copy(ew_hbm.at[pl.ds(base, CHUNK)], wv)

        load_idx(0, s0, d0, w0)
        pltpu.async_copy(g_hbm.at[s0], r0, sem0)

        def pair_body(g, _):
            load_idx(2 * g + 1, s1, d1, w1)
            pltpu.async_copy(g_hbm.at[s1], r1, sem1)
            pltpu.make_async_copy(g_hbm.at[s0], r0, sem0).wait()
            _scale_rows(r0, w0, CHUNK, d_valid)
            pltpu.sync_copy(r0, acc_sh.at[d0], add=True)

            @pl.when(g < PAIRS - 1)
            def _():
                load_idx(2 * g + 2, s0, d0, w0)
                pltpu.async_copy(g_hbm.at[s0], r0, sem0)
            pltpu.make_async_copy(g_hbm.at[s1], r1, sem1).wait()
            _scale_rows(r1, w1, CHUNK, d_valid)
            pltpu.sync_copy(r1, acc_sh.at[d1], add=True)
            return 0
        lax.fori_loop(0, PAIRS, pair_body, 0)

        tbase = ebase + FULL * CHUNK
        pltpu.sync_copy(src_hbm.at[pl.ds(tbase, TAIL)], ts)
        pltpu.sync_copy(dst_hbm.at[pl.ds(tbase, TAIL)], td)
        pltpu.sync_copy(ew_hbm.at[pl.ds(tbase, TAIL)], tw)
        pltpu.async_copy(g_hbm.at[ts], r0.at[pl.ds(0, TAIL)], sem0).wait()
        _scale_rows(r0, tw, TAIL, d_valid)
        pltpu.sync_copy(r0.at[pl.ds(0, TAIL)], acc_sh.at[td], add=True)

        plsc.subcore_barrier()

        for q in range(RITERS):
            rid = s + NS * q

            @pl.when(rid < N_FULL)
            def _():
                r_0 = rid * RCHUNK
                pltpu.sync_copy(acc_sh.at[pl.ds(r_0, RCHUNK)],
                                out_hbm.at[c, pl.ds(r_0, RCHUNK)])

        @pl.when(s == NS - 1)
        def _():
            pltpu.sync_copy(acc_sh.at[pl.ds(TAIL_BASE, TAIL_ROWS)],
                            out_hbm.at[c, pl.ds(TAIL_BASE, TAIL_ROWS)])

    return sc_scatter


def _sc_scatter_128(g, src, dst, ew):
    return _make_sc_scatter(128)(g, src, dst, ew)


def _sc_scatter_pad64(g, src, dst, ew):
    return _make_sc_scatter(64)(g, src, dst, ew)


@functools.cache
def _make_sc_gather_pq():
    mesh = plsc.VectorSubcoreMesh(core_axis_name="c", subcore_axis_name="s",
                                  num_cores=NC, num_subcores=NS)

    @functools.partial(
        pl.kernel,
        out_type=jax.ShapeDtypeStruct((E, D), jnp.float32),
        mesh=mesh,
        scratch_types=[
            pltpu.VMEM((CHUNK,), jnp.int32),
            pltpu.VMEM((CHUNK,), jnp.int32),
            pltpu.VMEM((CHUNK,), jnp.int32),
            pltpu.VMEM((CHUNK,), jnp.int32),
            pltpu.VMEM((CHUNK, D), jnp.float32),
            pltpu.VMEM((CHUNK, D), jnp.float32),
            pltpu.VMEM((CHUNK, D), jnp.float32),
            pltpu.VMEM((CHUNK, D), jnp.float32),
            pltpu.VMEM((TAIL,), jnp.int32),
            pltpu.VMEM((TAIL,), jnp.int32),
            pltpu.SemaphoreType.DMA,
            pltpu.SemaphoreType.DMA,
        ],
    )
    def sc_gather(p_hbm, q_hbm, src_hbm, dst_hbm, out_hbm,
                  s0, s1, d0, d1, p0, p1, q0, q1, ts, td, sem0, sem1):
        c = lax.axis_index("c")
        s = lax.axis_index("s")
        wid = s * NC + c
        ebase = wid * W_EDGES

        def load_idx(cid, sv, dv):
            base = ebase + cid * CHUNK
            pltpu.sync_copy(src_hbm.at[pl.ds(base, CHUNK)], sv)
            pltpu.sync_copy(dst_hbm.at[pl.ds(base, CHUNK)], dv)

        def start(sv, dv, pv, qv, sem):
            pltpu.async_copy(p_hbm.at[sv], pv, sem)
            pltpu.async_copy(q_hbm.at[dv], qv, sem)

        def finish(cid, sv, dv, pv, qv, sem):
            pltpu.make_async_copy(p_hbm.at[sv], pv, sem).wait()
            pltpu.make_async_copy(q_hbm.at[dv], qv, sem).wait()

            def add_body(row, _):
                for k in range(D // 16):
                    sl = pl.ds(k * 16, 16)
                    pv[row, sl] = pv[row, sl] + qv[row, sl]
                return 0
            lax.fori_loop(0, CHUNK, add_body, 0)
            base = ebase + cid * CHUNK
            pltpu.sync_copy(pv, out_hbm.at[pl.ds(base, CHUNK)])

        load_idx(0, s0, d0)
        start(s0, d0, p0, q0, sem0)

        def pair_body(g, _):
            load_idx(2 * g + 1, s1, d1)
            start(s1, d1, p1, q1, sem1)
            finish(2 * g, s0, d0, p0, q0, sem0)

            @pl.when(g < PAIRS - 1)
            def _():
                load_idx(2 * g + 2, s0, d0)
                start(s0, d0, p0, q0, sem0)
            finish(2 * g + 1, s1, d1, p1, q1, sem1)
            return 0
        lax.fori_loop(0, PAIRS, pair_body, 0)

        tbase = ebase + FULL * CHUNK
        pltpu.sync_copy(src_hbm.at[pl.ds(tbase, TAIL)], ts)
        pltpu.sync_copy(dst_hbm.at[pl.ds(tbase, TAIL)], td)
        cpp = pltpu.async_copy(p_hbm.at[ts], p0.at[pl.ds(0, TAIL)], sem0)
        cpq = pltpu.async_copy(q_hbm.at[td], q0.at[pl.ds(0, TAIL)], sem1)
        cpp.wait()
        cpq.wait()

        def tadd(row, _):
            for k in range(D // 16):
                sl = pl.ds(k * 16, 16)
                p0[row, sl] = p0[row, sl] + q0[row, sl]
            return 0
        lax.fori_loop(0, TAIL, tadd, 0)
        pltpu.sync_copy(p0.at[pl.ds(0, TAIL)], out_hbm.at[pl.ds(tbase, TAIL)])

    return sc_gather


def _sc_gather_pq(p, q, src, dst):
    return _make_sc_gather_pq()(p, q, src, dst)


_RT = 2000
_ET = 512


def _tc_dual_matmul(x, wa, wb):
    n, k = x.shape
    ma = wa.shape[1]
    mb = wb.shape[1]

    def body(x_ref, wa_ref, wb_ref, oa_ref, ob_ref):
        xv = x_ref[...]
        oa_ref[...] = jnp.dot(xv, wa_ref[...], preferred_element_type=jnp.float32)
        ob_ref[...] = jnp.dot(xv, wb_ref[...], preferred_element_type=jnp.float32)

    return pl.pallas_call(
        body,
        grid=(n // _RT,),
        in_specs=[
            pl.BlockSpec((_RT, k), lambda i: (i, 0)),
            pl.BlockSpec((k, ma), lambda i: (0, 0)),
            pl.BlockSpec((k, mb), lambda i: (0, 0)),
        ],
        out_specs=[
            pl.BlockSpec((_RT, ma), lambda i: (i, 0)),
            pl.BlockSpec((_RT, mb), lambda i: (i, 0)),
        ],
        out_shape=[
            jax.ShapeDtypeStruct((n, ma), jnp.float32),
            jax.ShapeDtypeStruct((n, mb), jnp.float32),
        ],
    )(x, wa, wb)


def _tc_stage(p, r, b, wa, wb, relu):
    n, m = r.shape
    ma = wa.shape[1]
    mb = wb.shape[1]

    def body(p_ref, r_ref, b_ref, wa_ref, wb_ref, ga_ref, gb_ref):
        h = p_ref[0, :, :m] + p_ref[1, :, :m] + r_ref[...] + b_ref[...]
        if relu:
            h = jnp.maximum(h, 0.0)
        ga_ref[...] = jnp.dot(h, wa_ref[...], preferred_element_type=jnp.float32)
        gb_ref[...] = jnp.dot(h, wb_ref[...], preferred_element_type=jnp.float32)

    return pl.pallas_call(
        body,
        grid=(n // _RT,),
        in_specs=[
            pl.BlockSpec((NC, _RT, D), lambda i: (0, i, 0)),
            pl.BlockSpec((_RT, m), lambda i: (i, 0)),
            pl.BlockSpec((1, m), lambda i: (0, 0)),
            pl.BlockSpec((m, ma), lambda i: (0, 0)),
            pl.BlockSpec((m, mb), lambda i: (0, 0)),
        ],
        out_specs=[
            pl.BlockSpec((_RT, ma), lambda i: (i, 0)),
            pl.BlockSpec((_RT, mb), lambda i: (i, 0)),
        ],
        out_shape=[
            jax.ShapeDtypeStruct((n, ma), jnp.float32),
            jax.ShapeDtypeStruct((n, mb), jnp.float32),
        ],
    )(p, r, b, wa, wb)


def _tc_decode(s, b1, w2_row, b2):
    def body(s_ref, b1_ref, w2_ref, b2_ref, o_ref):
        hid = jnp.maximum(s_ref[...] + b1_ref[...], 0.0)
        o_ref[...] = jnp.sum(hid * w2_ref[...], axis=1) + b2_ref[0, 0]

    return pl.pallas_call(
        body,
        grid=(E // _ET,),
        in_specs=[
            pl.BlockSpec((_ET, D), lambda i: (i, 0)),
            pl.BlockSpec((1, D), lambda i: (0, 0)),
            pl.BlockSpec((1, D), lambda i: (0, 0)),
            pl.BlockSpec((1, 1), lambda i: (0, 0)),
        ],
        out_specs=pl.BlockSpec((_ET,), lambda i: (i,)),
        out_shape=jax.ShapeDtypeStruct((E,), jnp.float32),
    )(s, b1, w2_row, b2)


def _pad_cols(w):
    k = w.shape[0]
    return jnp.concatenate(
        [w, jnp.zeros((k, D - w.shape[1]), jnp.float32)], axis=1)


def kernel(x, edge_index, edge_weight_norm,
           W1_rel, b1, W1_root,
           W2_rel, b2, W2_root,
           W3_rel, b3, W3_root,
           dec_W1, dec_b1, dec_W2, dec_b2):
    src = edge_index[0]
    dst = edge_index[1]
    ew = edge_weight_norm

    g1, r1 = _tc_dual_matmul(x, W1_rel, W1_root)
    p1 = _sc_scatter_128(g1, src, dst, ew)

    g2, r2 = _tc_stage(p1, r1, b1.reshape(1, -1),
                       _pad_cols(W2_rel), W2_root, relu=True)
    p2 = _sc_scatter_pad64(g2, src, dst, ew)

    g3, r3 = _tc_stage(p2, r2, b2.reshape(1, -1),
                       _pad_cols(W3_rel), W3_root, relu=True)
    p3 = _sc_scatter_pad64(g3, src, dst, ew)

    p_tab, q_tab = _tc_stage(p3, r3, b3.reshape(1, -1),
                             dec_W1[:D_Z], dec_W1[D_Z:], relu=False)

    s = _sc_gather_pq(p_tab, q_tab, src, dst)
    out = _tc_decode(s, dec_b1.reshape(1, -1),
                     dec_W2.reshape(1, -1), dec_b2.reshape(1, 1))
    return out


# speedup vs baseline: 5.5000x; 1.2134x over previous
import functools

import jax
import jax.numpy as jnp
from jax import lax
from jax.experimental import pallas as pl
from jax.experimental.pallas import tpu as pltpu
from jax.experimental.pallas import tpu_sc as plsc

N = 10000
E = 320000
D = 128
D_Z = 64

NC = 2
NS = 16
NW = NC * NS

CHUNK = 128
W_EDGES = E // NW
FULL = W_EDGES // CHUNK
PAIRS = FULL // 2
TAIL = W_EDGES - FULL * CHUNK

RCHUNK = 128
N_FULL = N // RCHUNK
TAIL_BASE = N_FULL * RCHUNK
TAIL_ROWS = N - TAIL_BASE
RITERS = -(-(N_FULL + 1) // NS)


def _zero_vmem_rows(buf, nrows, d):
    def body(i, _):
        for k in range(d // 16):
            buf[i, pl.ds(k * 16, 16)] = jnp.zeros((16,), jnp.float32)
        return 0
    lax.fori_loop(0, nrows, body, 0)


def _lane_splat(vec, lane):
    return lax.gather(
        vec,
        jnp.full((16, 1), lane, jnp.int32),
        dimension_numbers=lax.GatherDimensionNumbers(
            offset_dims=(), collapsed_slice_dims=(0,), start_index_map=(0,)),
        slice_sizes=(1,),
        mode=lax.GatherScatterMode.PROMISE_IN_BOUNDS)


def _scale_rows(rows, ew_ref, nedges, d_valid):
    def body(j, _):
        evec = ew_ref[pl.ds(j * 16, 16)]
        for e16 in range(16):
            bvec = _lane_splat(evec, e16)
            row = j * 16 + e16
            for k in range(d_valid // 16):
                sl = pl.ds(k * 16, 16)
                rows[row, sl] = rows[row, sl] * bvec
        return 0
    lax.fori_loop(0, nedges // 16, body, 0)


@functools.cache
def _make_sc_scatter(d_valid):
    mesh = plsc.VectorSubcoreMesh(core_axis_name="c", subcore_axis_name="s",
                                  num_cores=NC, num_subcores=NS)

    @functools.partial(
        pl.kernel,
        out_type=jax.ShapeDtypeStruct((NC, N, D), jnp.float32),
        mesh=mesh,
        scratch_types=[
            pltpu.VMEM((CHUNK,), jnp.int32),
            pltpu.VMEM((CHUNK,), jnp.int32),
            pltpu.VMEM((CHUNK,), jnp.int32),
            pltpu.VMEM((CHUNK,), jnp.int32),
            pltpu.VMEM((CHUNK,), jnp.float32),
            pltpu.VMEM((CHUNK,), jnp.float32),
            pltpu.VMEM((CHUNK, D), jnp.float32),
            pltpu.VMEM((CHUNK, D), jnp.float32),
            pltpu.VMEM((TAIL,), jnp.int32),
            pltpu.VMEM((TAIL,), jnp.int32),
            pltpu.VMEM((TAIL,), jnp.float32),
            pltpu.VMEM_SHARED((N, D), jnp.float32),
            pltpu.SemaphoreType.DMA,
            pltpu.SemaphoreType.DMA,
        ],
    )
    def sc_scatter(g_hbm, src_hbm, dst_hbm, ew_hbm, out_hbm,
                   s0, s1, d0, d1, w0, w1, r0, r1, ts, td, tw,
                   acc_sh, sem0, sem1):
        c = lax.axis_index("c")
        s = lax.axis_index("s")
        wid = s * NC + c
        ebase = wid * W_EDGES

        _zero_vmem_rows(r0, CHUNK, D)
        for q in range(RITERS):
            rid = s + NS * q

            @pl.when(rid < N_FULL)
            def _():
                pltpu.sync_copy(r0, acc_sh.at[pl.ds(rid * RCHUNK, RCHUNK)])

        @pl.when(s == NS - 1)
        def _():
            pltpu.sync_copy(r0.at[pl.ds(0, TAIL_ROWS)],
                            acc_sh.at[pl.ds(TAIL_BASE, TAIL_ROWS)])
        plsc.subcore_barrier()

        def load_idx(cid, sv, dv, wv):
            base = ebase + cid * CHUNK
            pltpu.sync_copy(src_hbm.at[pl.ds(base, CHUNK)], sv)
            pltpu.sync_copy(dst_hbm.at[pl.ds(base, CHUNK)], dv)
            pltpu.sync_copy(ew_hbm.at[pl.ds(base, CHUNK)], wv)

        load_idx(0, s0, d0, w0)
        pltpu.async_copy(g_hbm.at[s0], r0, sem0)

        def pair_body(g, _):
            load_idx(2 * g + 1, s1, d1, w1)
            pltpu.async_copy(g_hbm.at[s1], r1, sem1)
            pltpu.make_async_copy(g_hbm.at[s0], r0, sem0).wait()
            _scale_rows(r0, w0, CHUNK, d_valid)
            pltpu.sync_copy(r0, acc_sh.at[d0], add=True)

            @pl.when(g < PAIRS - 1)
            def _():
                load_idx(2 * g + 2, s0, d0, w0)
                pltpu.async_copy(g_hbm.at[s0], r0, sem0)
            pltpu.make_async_copy(g_hbm.at[s1], r1, sem1).wait()
            _scale_rows(r1, w1, CHUNK, d_valid)
            pltpu.sync_copy(r1, acc_sh.at[d1], add=True)
            return 0
        lax.fori_loop(0, PAIRS, pair_body, 0)

        tbase = ebase + FULL * CHUNK
        pltpu.sync_copy(src_hbm.at[pl.ds(tbase, TAIL)], ts)
        pltpu.sync_copy(dst_hbm.at[pl.ds(tbase, TAIL)], td)
        pltpu.sync_copy(ew_hbm.at[pl.ds(tbase, TAIL)], tw)
        pltpu.async_copy(g_hbm.at[ts], r0.at[pl.ds(0, TAIL)], sem0).wait()
        _scale_rows(r0, tw, TAIL, d_valid)
        pltpu.sync_copy(r0.at[pl.ds(0, TAIL)], acc_sh.at[td], add=True)

        plsc.subcore_barrier()

        for q in range(RITERS):
            rid = s + NS * q

            @pl.when(rid < N_FULL)
            def _():
                r_0 = rid * RCHUNK
                pltpu.sync_copy(acc_sh.at[pl.ds(r_0, RCHUNK)],
                                out_hbm.at[c, pl.ds(r_0, RCHUNK)])

        @pl.when(s == NS - 1)
        def _():
            pltpu.sync_copy(acc_sh.at[pl.ds(TAIL_BASE, TAIL_ROWS)],
                            out_hbm.at[c, pl.ds(TAIL_BASE, TAIL_ROWS)])

    return sc_scatter


def _sc_scatter_128(g, src, dst, ew):
    return _make_sc_scatter(128)(g, src, dst, ew)


def _sc_scatter_pad64(g, src, dst, ew):
    return _make_sc_scatter(64)(g, src, dst, ew)


@functools.cache
def _make_sc_gather_pq():
    mesh = plsc.VectorSubcoreMesh(core_axis_name="c", subcore_axis_name="s",
                                  num_cores=NC, num_subcores=NS)

    @functools.partial(
        pl.kernel,
        out_type=jax.ShapeDtypeStruct((E, D), jnp.float32),
        mesh=mesh,
        scratch_types=[
            pltpu.VMEM((CHUNK,), jnp.int32),
            pltpu.VMEM((CHUNK,), jnp.int32),
            pltpu.VMEM((CHUNK,), jnp.int32),
            pltpu.VMEM((CHUNK,), jnp.int32),
            pltpu.VMEM((CHUNK, D), jnp.float32),
            pltpu.VMEM((CHUNK, D), jnp.float32),
            pltpu.VMEM((CHUNK, D), jnp.float32),
            pltpu.VMEM((CHUNK, D), jnp.float32),
            pltpu.VMEM((TAIL,), jnp.int32),
            pltpu.VMEM((TAIL,), jnp.int32),
            pltpu.SemaphoreType.DMA,
            pltpu.SemaphoreType.DMA,
        ],
    )
    def sc_gather(p_hbm, q_hbm, src_hbm, dst_hbm, out_hbm,
                  s0, s1, d0, d1, p0, p1, q0, q1, ts, td, sem0, sem1):
        c = lax.axis_index("c")
        s = lax.axis_index("s")
        wid = s * NC + c
        ebase = wid * W_EDGES

        def load_idx(cid, sv, dv):
            base = ebase + cid * CHUNK
            pltpu.sync_copy(src_hbm.at[pl.ds(base, CHUNK)], sv)
            pltpu.sync_copy(dst_hbm.at[pl.ds(base, CHUNK)], dv)

        def start(sv, dv, pv, qv, sem):
            pltpu.async_copy(p_hbm.at[sv], pv, sem)
            pltpu.async_copy(q_hbm.at[dv], qv, sem)

        def finish(cid, sv, dv, pv, qv, sem):
            pltpu.make_async_copy(p_hbm.at[sv], pv, sem).wait()
            pltpu.make_async_copy(q_hbm.at[dv], qv, sem).wait()

            def add_body(row, _):
                for k in range(D // 16):
                    sl = pl.ds(k * 16, 16)
                    pv[row, sl] = pv[row, sl] + qv[row, sl]
                return 0
            lax.fori_loop(0, CHUNK, add_body, 0)
            base = ebase + cid * CHUNK
            pltpu.sync_copy(pv, out_hbm.at[pl.ds(base, CHUNK)])

        load_idx(0, s0, d0)
        start(s0, d0, p0, q0, sem0)

        def pair_body(g, _):
            load_idx(2 * g + 1, s1, d1)
            start(s1, d1, p1, q1, sem1)
            finish(2 * g, s0, d0, p0, q0, sem0)

            @pl.when(g < PAIRS - 1)
            def _():
                load_idx(2 * g + 2, s0, d0)
                start(s0, d0, p0, q0, sem0)
            finish(2 * g + 1, s1, d1, p1, q1, sem1)
            return 0
        lax.fori_loop(0, PAIRS, pair_body, 0)

        tbase = ebase + FULL * CHUNK
        pltpu.sync_copy(src_hbm.at[pl.ds(tbase, TAIL)], ts)
        pltpu.sync_copy(dst_hbm.at[pl.ds(tbase, TAIL)], td)
        cpp = pltpu.async_copy(p_hbm.at[ts], p0.at[pl.ds(0, TAIL)], sem0)
        cpq = pltpu.async_copy(q_hbm.at[td], q0.at[pl.ds(0, TAIL)], sem1)
        cpp.wait()
        cpq.wait()

        def tadd(row, _):
            for k in range(D // 16):
                sl = pl.ds(k * 16, 16)
                p0[row, sl] = p0[row, sl] + q0[row, sl]
            return 0
        lax.fori_loop(0, TAIL, tadd, 0)
        pltpu.sync_copy(p0.at[pl.ds(0, TAIL)], out_hbm.at[pl.ds(tbase, TAIL)])

    return sc_gather


def _sc_gather_pq(p, q, src, dst):
    return _make_sc_gather_pq()(p, q, src, dst)


_RT = 2000
_ET = 4000


def _tc_root(x, w):
    n, k = x.shape
    m = w.shape[1]

    def body(x_ref, w_ref, o_ref):
        o_ref[...] = jnp.dot(x_ref[...], w_ref[...],
                             preferred_element_type=jnp.float32)

    return pl.pallas_call(
        body,
        grid=(n // _RT,),
        in_specs=[
            pl.BlockSpec((_RT, k), lambda i: (i, 0)),
            pl.BlockSpec((k, m), lambda i: (0, 0)),
        ],
        out_specs=pl.BlockSpec((_RT, m), lambda i: (i, 0)),
        out_shape=jax.ShapeDtypeStruct((n, m), jnp.float32),
    )(x, w)


def _tc_stage(p, r, b, w_rel, w_next, m_in, pad_out):
    n = r.shape[0]
    mo = w_rel.shape[1]
    mb = w_next.shape[1]
    ma = D if pad_out else mo

    def body(p_ref, r_ref, b_ref, wr_ref, wn_ref, h_ref, o_ref):
        agg = p_ref[0, :, :m_in] + p_ref[1, :, :m_in]
        h = jnp.maximum(
            jnp.dot(agg, wr_ref[...], preferred_element_type=jnp.float32)
            + r_ref[...] + b_ref[...], 0.0)
        if pad_out:
            h_ref[...] = jnp.concatenate(
                [h, jnp.zeros((h.shape[0], D - mo), jnp.float32)], axis=1)
        else:
            h_ref[...] = h
        o_ref[...] = jnp.dot(h, wn_ref[...], preferred_element_type=jnp.float32)

    return pl.pallas_call(
        body,
        grid=(n // _RT,),
        in_specs=[
            pl.BlockSpec((NC, _RT, D), lambda i: (0, i, 0)),
            pl.BlockSpec((_RT, mo), lambda i: (i, 0)),
            pl.BlockSpec((1, mo), lambda i: (0, 0)),
            pl.BlockSpec((m_in, mo), lambda i: (0, 0)),
            pl.BlockSpec((mo, mb), lambda i: (0, 0)),
        ],
        out_specs=[
            pl.BlockSpec((_RT, ma), lambda i: (i, 0)),
            pl.BlockSpec((_RT, mb), lambda i: (i, 0)),
        ],
        out_shape=[
            jax.ShapeDtypeStruct((n, ma), jnp.float32),
            jax.ShapeDtypeStruct((n, mb), jnp.float32),
        ],
    )(p, r, b, w_rel, w_next)


def _tc_stage_z(p, r, b, w_rel, wa, wb):
    n = r.shape[0]

    def body(p_ref, r_ref, b_ref, wr_ref, wa_ref, wb_ref, pa_ref, qb_ref):
        agg = p_ref[0, :, :D_Z] + p_ref[1, :, :D_Z]
        z = (jnp.dot(agg, wr_ref[...], preferred_element_type=jnp.float32)
             + r_ref[...] + b_ref[...])
        pa_ref[...] = jnp.dot(z, wa_ref[...], preferred_element_type=jnp.float32)
        qb_ref[...] = jnp.dot(z, wb_ref[...], preferred_element_type=jnp.float32)

    return pl.pallas_call(
        body,
        grid=(n // _RT,),
        in_specs=[
            pl.BlockSpec((NC, _RT, D), lambda i: (0, i, 0)),
            pl.BlockSpec((_RT, D_Z), lambda i: (i, 0)),
            pl.BlockSpec((1, D_Z), lambda i: (0, 0)),
            pl.BlockSpec((D_Z, D_Z), lambda i: (0, 0)),
            pl.BlockSpec((D_Z, D), lambda i: (0, 0)),
            pl.BlockSpec((D_Z, D), lambda i: (0, 0)),
        ],
        out_specs=[
            pl.BlockSpec((_RT, D), lambda i: (i, 0)),
            pl.BlockSpec((_RT, D), lambda i: (i, 0)),
        ],
        out_shape=[
            jax.ShapeDtypeStruct((n, D), jnp.float32),
            jax.ShapeDtypeStruct((n, D), jnp.float32),
        ],
    )(p, r, b, w_rel, wa, wb)


def _tc_decode(s, b1, w2, b2):
    def body(s_ref, b1_ref, w2_ref, b2_ref, o_ref):
        hid = jnp.maximum(s_ref[...] + b1_ref[...], 0.0)
        o_ref[...] = (jnp.dot(hid, w2_ref[...],
                              preferred_element_type=jnp.float32)
                      + b2_ref[0, 0])

    return pl.pallas_call(
        body,
        grid=(E // _ET,),
        in_specs=[
            pl.BlockSpec((_ET, D), lambda i: (i, 0)),
            pl.BlockSpec((1, D), lambda i: (0, 0)),
            pl.BlockSpec((D, 1), lambda i: (0, 0)),
            pl.BlockSpec((1, 1), lambda i: (0, 0)),
        ],
        out_specs=pl.BlockSpec((_ET, 1), lambda i: (i, 0)),
        out_shape=jax.ShapeDtypeStruct((E, 1), jnp.float32),
    )(s, b1, w2, b2)


def kernel(x, edge_index, edge_weight_norm,
           W1_rel, b1, W1_root,
           W2_rel, b2, W2_root,
           W3_rel, b3, W3_root,
           dec_W1, dec_b1, dec_W2, dec_b2):
    src = edge_index[0]
    dst = edge_index[1]
    ew = edge_weight_norm

    r1 = _tc_root(x, W1_root)
    p1 = _sc_scatter_128(x, src, dst, ew)
    h1, r2 = _tc_stage(p1, r1, b1.reshape(1, -1), W1_rel, W2_root,
                       m_in=D, pad_out=False)

    p2 = _sc_scatter_128(h1, src, dst, ew)
    h2p, r3 = _tc_stage(p2, r2, b2.reshape(1, -1), W2_rel, W3_root,
                        m_in=D, pad_out=True)

    p3 = _sc_scatter_pad64(h2p, src, dst, ew)
    p_tab, q_tab = _tc_stage_z(p3, r3, b3.reshape(1, -1), W3_rel,
                               dec_W1[:D_Z], dec_W1[D_Z:])

    s = _sc_gather_pq(p_tab, q_tab, src, dst)
    out = _tc_decode(s, dec_b1.reshape(1, -1),
                     dec_W2, dec_b2.reshape(1, 1))
    return out[:, 0]


# speedup vs baseline: 6.9829x; 1.2696x over previous
import functools

import jax
import jax.numpy as jnp
from jax import lax
from jax.experimental import pallas as pl
from jax.experimental.pallas import tpu as pltpu
from jax.experimental.pallas import tpu_sc as plsc

N = 10000
E = 320000
D = 128
D_Z = 64

NC = 2
NS = 16
NW = NC * NS

CHUNK = 128
W_EDGES = E // NW
FULL = W_EDGES // CHUNK
PAIRS = FULL // 2
TAIL = W_EDGES - FULL * CHUNK

RCHUNK = 128
N_FULL = N // RCHUNK
TAIL_BASE = N_FULL * RCHUNK
TAIL_ROWS = N - TAIL_BASE
RITERS = -(-(N_FULL + 1) // NS)


def _zero_vmem_rows(buf, nrows, d):
    def body(i, _):
        for k in range(d // 16):
            buf[i, pl.ds(k * 16, 16)] = jnp.zeros((16,), jnp.float32)
        return 0
    lax.fori_loop(0, nrows, body, 0)


def _lane_splat(vec, lane):
    return lax.gather(
        vec,
        jnp.full((16, 1), lane, jnp.int32),
        dimension_numbers=lax.GatherDimensionNumbers(
            offset_dims=(), collapsed_slice_dims=(0,), start_index_map=(0,)),
        slice_sizes=(1,),
        mode=lax.GatherScatterMode.PROMISE_IN_BOUNDS)


def _scale_rows(rows, ew_ref, nedges, d_valid):
    def body(j, _):
        evec = ew_ref[pl.ds(j * 16, 16)]
        for e16 in range(16):
            bvec = _lane_splat(evec, e16)
            row = j * 16 + e16
            for k in range(d_valid // 16):
                sl = pl.ds(k * 16, 16)
                rows[row, sl] = rows[row, sl] * bvec
        return 0
    lax.fori_loop(0, nedges // 16, body, 0)


@functools.cache
def _make_sc_scatter(d_valid):
    mesh = plsc.VectorSubcoreMesh(core_axis_name="c", subcore_axis_name="s",
                                  num_cores=NC, num_subcores=NS)

    @functools.partial(
        pl.kernel,
        out_type=jax.ShapeDtypeStruct((NC, N, D), jnp.float32),
        mesh=mesh,
        scratch_types=[
            pltpu.VMEM((CHUNK,), jnp.int32),
            pltpu.VMEM((CHUNK,), jnp.int32),
            pltpu.VMEM((CHUNK,), jnp.int32),
            pltpu.VMEM((CHUNK,), jnp.int32),
            pltpu.VMEM((CHUNK,), jnp.float32),
            pltpu.VMEM((CHUNK,), jnp.float32),
            pltpu.VMEM((CHUNK, D), jnp.float32),
            pltpu.VMEM((CHUNK, D), jnp.float32),
            pltpu.VMEM((TAIL,), jnp.int32),
            pltpu.VMEM((TAIL,), jnp.int32),
            pltpu.VMEM((TAIL,), jnp.float32),
            pltpu.VMEM_SHARED((N, D), jnp.float32),
            pltpu.SemaphoreType.DMA,
            pltpu.SemaphoreType.DMA,
            pltpu.SemaphoreType.DMA,
            pltpu.SemaphoreType.DMA,
            pltpu.SemaphoreType.DMA,
            pltpu.SemaphoreType.DMA,
        ],
    )
    def sc_scatter(g_hbm, src_hbm, dst_hbm, ew_hbm, out_hbm,
                   s0, s1, d0, d1, w0, w1, r0, r1, ts, td, tw,
                   acc_sh, sem0, sem1, si0, si1, sj0, sj1):
        c = lax.axis_index("c")
        s = lax.axis_index("s")
        wid = s * NC + c
        ebase = wid * W_EDGES

        _zero_vmem_rows(r0, CHUNK, D)
        for q in range(RITERS):
            rid = s + NS * q

            @pl.when(rid < N_FULL)
            def _():
                pltpu.sync_copy(r0, acc_sh.at[pl.ds(rid * RCHUNK, RCHUNK)])

        @pl.when(s == NS - 1)
        def _():
            pltpu.sync_copy(r0.at[pl.ds(0, TAIL_ROWS)],
                            acc_sh.at[pl.ds(TAIL_BASE, TAIL_ROWS)])
        plsc.subcore_barrier()

        def issue_idx(cid, sv, dv, wv, semi, semj):
            base = ebase + cid * CHUNK
            pltpu.async_copy(src_hbm.at[pl.ds(base, CHUNK)], sv, semi)
            pltpu.async_copy(dst_hbm.at[pl.ds(base, CHUNK)], dv, semj)
            pltpu.async_copy(ew_hbm.at[pl.ds(base, CHUNK)], wv, semj)

        def start_gather(cid, sv, dv, wv, rv, semi, sem):
            base = ebase + cid * CHUNK
            pltpu.make_async_copy(src_hbm.at[pl.ds(base, CHUNK)], sv,
                                  semi).wait()
            pltpu.async_copy(g_hbm.at[sv], rv, sem)

        def finish(cid, sv, dv, wv, rv, semj, sem):
            base = ebase + cid * CHUNK
            pltpu.make_async_copy(g_hbm.at[sv], rv, sem).wait()
            pltpu.make_async_copy(dst_hbm.at[pl.ds(base, CHUNK)], dv,
                                  semj).wait()
            pltpu.make_async_copy(ew_hbm.at[pl.ds(base, CHUNK)], wv,
                                  semj).wait()
            _scale_rows(rv, wv, CHUNK, d_valid)
            pltpu.sync_copy(rv, acc_sh.at[dv], add=True)

        issue_idx(0, s0, d0, w0, si0, sj0)
        start_gather(0, s0, d0, w0, r0, si0, sem0)

        def pair_body(g, _):
            issue_idx(2 * g + 1, s1, d1, w1, si1, sj1)
            start_gather(2 * g + 1, s1, d1, w1, r1, si1, sem1)
            finish(2 * g, s0, d0, w0, r0, sj0, sem0)

            @pl.when(g < PAIRS - 1)
            def _():
                issue_idx(2 * g + 2, s0, d0, w0, si0, sj0)
                start_gather(2 * g + 2, s0, d0, w0, r0, si0, sem0)
            finish(2 * g + 1, s1, d1, w1, r1, sj1, sem1)
            return 0
        lax.fori_loop(0, PAIRS, pair_body, 0)

        tbase = ebase + FULL * CHUNK
        pltpu.sync_copy(src_hbm.at[pl.ds(tbase, TAIL)], ts)
        pltpu.sync_copy(dst_hbm.at[pl.ds(tbase, TAIL)], td)
        pltpu.sync_copy(ew_hbm.at[pl.ds(tbase, TAIL)], tw)
        pltpu.async_copy(g_hbm.at[ts], r0.at[pl.ds(0, TAIL)], sem0).wait()
        _scale_rows(r0, tw, TAIL, d_valid)
        pltpu.sync_copy(r0.at[pl.ds(0, TAIL)], acc_sh.at[td], add=True)

        plsc.subcore_barrier()

        for q in range(RITERS):
            rid = s + NS * q

            @pl.when(rid < N_FULL)
            def _():
                r_0 = rid * RCHUNK
                pltpu.sync_copy(acc_sh.at[pl.ds(r_0, RCHUNK)],
                                out_hbm.at[c, pl.ds(r_0, RCHUNK)])

        @pl.when(s == NS - 1)
        def _():
            pltpu.sync_copy(acc_sh.at[pl.ds(TAIL_BASE, TAIL_ROWS)],
                            out_hbm.at[c, pl.ds(TAIL_BASE, TAIL_ROWS)])

    return sc_scatter


def _sc_scatter_128(g, src, dst, ew):
    return _make_sc_scatter(128)(g, src, dst, ew)


def _sc_scatter_pad64(g, src, dst, ew):
    return _make_sc_scatter(64)(g, src, dst, ew)


@functools.cache
def _make_sc_gather_pq():
    mesh = plsc.VectorSubcoreMesh(core_axis_name="c", subcore_axis_name="s",
                                  num_cores=NC, num_subcores=NS)

    @functools.partial(
        pl.kernel,
        out_type=jax.ShapeDtypeStruct((E, D), jnp.float32),
        mesh=mesh,
        scratch_types=[
            pltpu.VMEM((CHUNK,), jnp.int32),
            pltpu.VMEM((CHUNK,), jnp.int32),
            pltpu.VMEM((CHUNK,), jnp.int32),
            pltpu.VMEM((CHUNK,), jnp.int32),
            pltpu.VMEM((CHUNK, D), jnp.float32),
            pltpu.VMEM((CHUNK, D), jnp.float32),
            pltpu.VMEM((CHUNK, D), jnp.float32),
            pltpu.VMEM((CHUNK, D), jnp.float32),
            pltpu.VMEM((TAIL,), jnp.int32),
            pltpu.VMEM((TAIL,), jnp.int32),
            pltpu.SemaphoreType.DMA,
            pltpu.SemaphoreType.DMA,
            pltpu.SemaphoreType.DMA,
            pltpu.SemaphoreType.DMA,
        ],
    )
    def sc_gather(p_hbm, q_hbm, src_hbm, dst_hbm, out_hbm,
                  s0, s1, d0, d1, p0, p1, q0, q1, ts, td,
                  sem0, sem1, si0, si1):
        c = lax.axis_index("c")
        s = lax.axis_index("s")
        wid = s * NC + c
        ebase = wid * W_EDGES

        def issue_idx(cid, sv, dv, semi):
            base = ebase + cid * CHUNK
            pltpu.async_copy(src_hbm.at[pl.ds(base, CHUNK)], sv, semi)
            pltpu.async_copy(dst_hbm.at[pl.ds(base, CHUNK)], dv, semi)

        def start(cid, sv, dv, pv, qv, semi, sem):
            base = ebase + cid * CHUNK
            pltpu.make_async_copy(src_hbm.at[pl.ds(base, CHUNK)], sv,
                                  semi).wait()
            pltpu.make_async_copy(dst_hbm.at[pl.ds(base, CHUNK)], dv,
                                  semi).wait()
            pltpu.async_copy(p_hbm.at[sv], pv, sem)
            pltpu.async_copy(q_hbm.at[dv], qv, sem)

        def finish(cid, sv, dv, pv, qv, sem):
            pltpu.make_async_copy(p_hbm.at[sv], pv, sem).wait()
            pltpu.make_async_copy(q_hbm.at[dv], qv, sem).wait()

            def add_body(row, _):
                for k in range(D // 16):
                    sl = pl.ds(k * 16, 16)
                    pv[row, sl] = pv[row, sl] + qv[row, sl]
                return 0
            lax.fori_loop(0, CHUNK, add_body, 0)
            base = ebase + cid * CHUNK
            pltpu.sync_copy(pv, out_hbm.at[pl.ds(base, CHUNK)])

        issue_idx(0, s0, d0, si0)
        start(0, s0, d0, p0, q0, si0, sem0)

        def pair_body(g, _):
            issue_idx(2 * g + 1, s1, d1, si1)
            start(2 * g + 1, s1, d1, p1, q1, si1, sem1)
            finish(2 * g, s0, d0, p0, q0, sem0)

            @pl.when(g < PAIRS - 1)
            def _():
                issue_idx(2 * g + 2, s0, d0, si0)
                start(2 * g + 2, s0, d0, p0, q0, si0, sem0)
            finish(2 * g + 1, s1, d1, p1, q1, sem1)
            return 0
        lax.fori_loop(0, PAIRS, pair_body, 0)

        tbase = ebase + FULL * CHUNK
        pltpu.sync_copy(src_hbm.at[pl.ds(tbase, TAIL)], ts)
        pltpu.sync_copy(dst_hbm.at[pl.ds(tbase, TAIL)], td)
        cpp = pltpu.async_copy(p_hbm.at[ts], p0.at[pl.ds(0, TAIL)], sem0)
        cpq = pltpu.async_copy(q_hbm.at[td], q0.at[pl.ds(0, TAIL)], sem1)
        cpp.wait()
        cpq.wait()

        def tadd(row, _):
            for k in range(D // 16):
                sl = pl.ds(k * 16, 16)
                p0[row, sl] = p0[row, sl] + q0[row, sl]
            return 0
        lax.fori_loop(0, TAIL, tadd, 0)
        pltpu.sync_copy(p0.at[pl.ds(0, TAIL)], out_hbm.at[pl.ds(tbase, TAIL)])

    return sc_gather


def _sc_gather_pq(p, q, src, dst):
    return _make_sc_gather_pq()(p, q, src, dst)


_RT = 2000
_ET = 4000


def _tc_root(x, w):
    n, k = x.shape
    m = w.shape[1]

    def body(x_ref, w_ref, o_ref):
        o_ref[...] = jnp.dot(x_ref[...], w_ref[...],
                             preferred_element_type=jnp.float32)

    return pl.pallas_call(
        body,
        grid=(n // _RT,),
        in_specs=[
            pl.BlockSpec((_RT, k), lambda i: (i, 0)),
            pl.BlockSpec((k, m), lambda i: (0, 0)),
        ],
        out_specs=pl.BlockSpec((_RT, m), lambda i: (i, 0)),
        out_shape=jax.ShapeDtypeStruct((n, m), jnp.float32),
    )(x, w)


def _tc_stage(p, r, b, w_rel, w_next, m_in, pad_out):
    n = r.shape[0]
    mo = w_rel.shape[1]
    mb = w_next.shape[1]
    ma = D if pad_out else mo

    def body(p_ref, r_ref, b_ref, wr_ref, wn_ref, h_ref, o_ref):
        agg = p_ref[0, :, :m_in] + p_ref[1, :, :m_in]
        h = jnp.maximum(
            jnp.dot(agg, wr_ref[...], preferred_element_type=jnp.float32)
            + r_ref[...] + b_ref[...], 0.0)
        if pad_out:
            h_ref[...] = jnp.concatenate(
                [h, jnp.zeros((h.shape[0], D - mo), jnp.float32)], axis=1)
        else:
            h_ref[...] = h
        o_ref[...] = jnp.dot(h, wn_ref[...], preferred_element_type=jnp.float32)

    return pl.pallas_call(
        body,
        grid=(n // _RT,),
        in_specs=[
            pl.BlockSpec((NC, _RT, D), lambda i: (0, i, 0)),
            pl.BlockSpec((_RT, mo), lambda i: (i, 0)),
            pl.BlockSpec((1, mo), lambda i: (0, 0)),
            pl.BlockSpec((m_in, mo), lambda i: (0, 0)),
            pl.BlockSpec((mo, mb), lambda i: (0, 0)),
        ],
        out_specs=[
            pl.BlockSpec((_RT, ma), lambda i: (i, 0)),
            pl.BlockSpec((_RT, mb), lambda i: (i, 0)),
        ],
        out_shape=[
            jax.ShapeDtypeStruct((n, ma), jnp.float32),
            jax.ShapeDtypeStruct((n, mb), jnp.float32),
        ],
    )(p, r, b, w_rel, w_next)


def _tc_stage_z(p, r, b, w_rel, wa, wb):
    n = r.shape[0]

    def body(p_ref, r_ref, b_ref, wr_ref, wa_ref, wb_ref, pa_ref, qb_ref):
        agg = p_ref[0, :, :D_Z] + p_ref[1, :, :D_Z]
        z = (jnp.dot(agg, wr_ref[...], preferred_element_type=jnp.float32)
             + r_ref[...] + b_ref[...])
        pa_ref[...] = jnp.dot(z, wa_ref[...], preferred_element_type=jnp.float32)
        qb_ref[...] = jnp.dot(z, wb_ref[...], preferred_element_type=jnp.float32)

    return pl.pallas_call(
        body,
        grid=(n // _RT,),
        in_specs=[
            pl.BlockSpec((NC, _RT, D), lambda i: (0, i, 0)),
            pl.BlockSpec((_RT, D_Z), lambda i: (i, 0)),
            pl.BlockSpec((1, D_Z), lambda i: (0, 0)),
            pl.BlockSpec((D_Z, D_Z), lambda i: (0, 0)),
            pl.BlockSpec((D_Z, D), lambda i: (0, 0)),
            pl.BlockSpec((D_Z, D), lambda i: (0, 0)),
        ],
        out_specs=[
            pl.BlockSpec((_RT, D), lambda i: (i, 0)),
            pl.BlockSpec((_RT, D), lambda i: (i, 0)),
        ],
        out_shape=[
            jax.ShapeDtypeStruct((n, D), jnp.float32),
            jax.ShapeDtypeStruct((n, D), jnp.float32),
        ],
    )(p, r, b, w_rel, wa, wb)


def _tc_decode(s, b1, w2, b2):
    def body(s_ref, b1_ref, w2_ref, b2_ref, o_ref):
        hid = jnp.maximum(s_ref[...] + b1_ref[...], 0.0)
        o_ref[...] = (jnp.dot(hid, w2_ref[...],
                              preferred_element_type=jnp.float32)
                      + b2_ref[0, 0])

    return pl.pallas_call(
        body,
        grid=(E // _ET,),
        in_specs=[
            pl.BlockSpec((_ET, D), lambda i: (i, 0)),
            pl.BlockSpec((1, D), lambda i: (0, 0)),
            pl.BlockSpec((D, 1), lambda i: (0, 0)),
            pl.BlockSpec((1, 1), lambda i: (0, 0)),
        ],
        out_specs=pl.BlockSpec((_ET, 1), lambda i: (i, 0)),
        out_shape=jax.ShapeDtypeStruct((E, 1), jnp.float32),
    )(s, b1, w2, b2)


def kernel(x, edge_index, edge_weight_norm,
           W1_rel, b1, W1_root,
           W2_rel, b2, W2_root,
           W3_rel, b3, W3_root,
           dec_W1, dec_b1, dec_W2, dec_b2):
    src = edge_index[0]
    dst = edge_index[1]
    ew = edge_weight_norm

    r1 = _tc_root(x, W1_root)
    p1 = _sc_scatter_128(x, src, dst, ew)
    h1, r2 = _tc_stage(p1, r1, b1.reshape(1, -1), W1_rel, W2_root,
                       m_in=D, pad_out=False)

    p2 = _sc_scatter_128(h1, src, dst, ew)
    h2p, r3 = _tc_stage(p2, r2, b2.reshape(1, -1), W2_rel, W3_root,
                        m_in=D, pad_out=True)

    p3 = _sc_scatter_pad64(h2p, src, dst, ew)
    p_tab, q_tab = _tc_stage_z(p3, r3, b3.reshape(1, -1), W3_rel,
                               dec_W1[:D_Z], dec_W1[D_Z:])

    s = _sc_gather_pq(p_tab, q_tab, src, dst)
    out = _tc_decode(s, dec_b1.reshape(1, -1),
                     dec_W2, dec_b2.reshape(1, 1))
    return out[:, 0]
